# Initial kernel scaffold; baseline (speedup 1.0000x reference)
import jax
import jax.numpy as jnp
from jax.experimental import pallas as pl


def kernel(x, hyperedge_index, W1, b1, gamma, beta, W2, b2):
    raise NotImplementedError("write your pallas kernel here")


# speedup vs baseline: 4.2265x; 4.2265x over previous
import functools

import jax
import jax.numpy as jnp
from jax import lax
from jax.experimental import pallas as pl
from jax.experimental.pallas import tpu as pltpu
from jax.experimental.pallas import tpu_sc as plsc

N = 10000
NE = 10000
P = 320000
D_FEAT = 128
HID = 256
OUT = 64
ALPHA = 0.1
STEPS = 10

NC, NS = 2, 16
NW = NC * NS
PPW = P // NW
CHUNK = 128
NCHUNK = 80
SLOTS = NCHUNK * CHUNK
ACC_ROWS = N + 112
RPT = ACC_ROWS // NS


def _sc_pass(feat):
    mesh = plsc.VectorSubcoreMesh(
        core_axis_name="c", subcore_axis_name="s", num_cores=NC, num_subcores=NS
    )

    @functools.partial(
        pl.kernel,
        out_type=jax.ShapeDtypeStruct((NC, ACC_ROWS, feat), jnp.float32),
        mesh=mesh,
        compiler_params=pltpu.CompilerParams(use_tc_tiling_on_sc=(feat == D_FEAT)),
        scratch_types=[
            pltpu.VMEM((NCHUNK, CHUNK), jnp.int32),
            pltpu.VMEM((NCHUNK, CHUNK), jnp.int32),
            pltpu.VMEM((CHUNK, feat), jnp.float32),
            pltpu.VMEM_SHARED((ACC_ROWS, feat), jnp.float32),
            pltpu.SemaphoreType.DMA,
        ],
    )
    def k(src, gidx, sidx, zeros, out, gidx_v, sidx_v, rows_v, acc, sem):
        c = lax.axis_index("c")
        s = lax.axis_index("s")
        wid = c * NS + s
        base = s * RPT
        off = 0
        for nrows in (128, 128, 128, 128, 120):
            pltpu.sync_copy(zeros.at[pl.ds(0, nrows)], acc.at[pl.ds(base + off, nrows)])
            off += nrows
        pltpu.sync_copy(gidx.at[wid], gidx_v)
        pltpu.sync_copy(sidx.at[wid], sidx_v)
        plsc.subcore_barrier()

        @pl.loop(0, NCHUNK)
        def _(j):
            pltpu.async_copy(src.at[gidx_v.at[j]], rows_v, sem).wait()
            pltpu.sync_copy(rows_v, acc.at[sidx_v.at[j]], add=True)

        plsc.subcore_barrier()
        pltpu.sync_copy(acc.at[pl.ds(base, RPT)], out.at[c, pl.ds(base, RPT)])

    return k


_pass16 = _sc_pass(16)
_pass128 = _sc_pass(D_FEAT)


def _prep_body(dp_ref, dep_ref, x_ref, rsqd_ref, rde_ref, xs_ref):
    d = dp_ref[0, :N, 0:1] + dp_ref[1, :N, 0:1]
    de = dep_ref[0, :N, 0:1] + dep_ref[1, :N, 0:1]
    rsqd = lax.rsqrt(d)
    rsqd_ref[...] = rsqd
    rde_ref[...] = 1.0 / de
    xs_ref[...] = x_ref[...] * rsqd


def _prep(dp, dep, x):
    return pl.pallas_call(
        _prep_body,
        out_shape=(
            jax.ShapeDtypeStruct((N, 1), jnp.float32),
            jax.ShapeDtypeStruct((N, 1), jnp.float32),
            jax.ShapeDtypeStruct((N, D_FEAT), jnp.float32),
        ),
    )(dp, dep, x)


def _z_body(xep_ref, rde_ref, z_ref, c_ref):
    z = (xep_ref[0, :N, :] + xep_ref[1, :N, :]) * rde_ref[...]
    z_ref[...] = z
    ssq = jnp.sum(z * z)
    c_ref[...] = jnp.full((1, 1), 0.5, jnp.float32) * lax.rsqrt(ssq)


def _tcz(xep, rde):
    return pl.pallas_call(
        _z_body,
        out_shape=(
            jax.ShapeDtypeStruct((N, D_FEAT), jnp.float32),
            jax.ShapeDtypeStruct((1, 1), jnp.float32),
        ),
    )(xep, rde)


def _g_body(xvp_ref, rsqd_ref, x_ref, cprev_ref, cinit_ref, g_ref, gs_ref):
    hv = (xvp_ref[0, :N, :] + xvp_ref[1, :N, :]) * rsqd_ref[...]
    g = ((1.0 - ALPHA) * cprev_ref[0:1, 0:1]) * hv + (ALPHA * cinit_ref[0:1, 0:1]) * x_ref[...]
    g_ref[...] = g
    gs_ref[...] = g * rsqd_ref[...]


def _tcg(xvp, rsqd, x, cprev, cinit):
    return pl.pallas_call(
        _g_body,
        out_shape=(
            jax.ShapeDtypeStruct((N, D_FEAT), jnp.float32),
            jax.ShapeDtypeStruct((N, D_FEAT), jnp.float32),
        ),
    )(xvp, rsqd, x, cprev, cinit)


def _mlp_body(g_ref, c_ref, w1_ref, b1_ref, gamma_ref, beta_ref, w2_ref, b2_ref, o_ref):
    f = g_ref[...] * c_ref[0:1, 0:1]
    h = jnp.dot(f, w1_ref[...], preferred_element_type=jnp.float32) + b1_ref[...]
    mu = jnp.mean(h, axis=-1, keepdims=True)
    var = jnp.mean((h - mu) * (h - mu), axis=-1, keepdims=True)
    hn = (h - mu) * lax.rsqrt(var + 1e-5) * gamma_ref[...] + beta_ref[...]
    hr = jnp.maximum(hn, 0.0)
    o_ref[...] = jnp.dot(hr, w2_ref[...], preferred_element_type=jnp.float32) + b2_ref[...]


def _mlp(g, c, w1, b1, gamma, beta, w2, b2):
    return pl.pallas_call(
        _mlp_body,
        out_shape=jax.ShapeDtypeStruct((N, OUT), jnp.float32),
    )(g, c, w1, b1, gamma, beta, w2, b2)


def _pad_idx(idx, pad_vals):
    blocks = idx.reshape(NW, PPW)
    pad = jnp.broadcast_to(pad_vals[:, None], (NW, SLOTS - PPW)).astype(jnp.int32)
    return jnp.concatenate([blocks, pad], axis=1).reshape(NW, NCHUNK, CHUNK)


def kernel(x, hyperedge_index, W1, b1, gamma, beta, W2, b2):
    V = hyperedge_index[0]
    E = hyperedge_index[1]
    gpad = jnp.zeros((NW,), jnp.int32)
    spad = N + (jnp.arange(NW, dtype=jnp.int32) % NS)
    Vg, Vs = _pad_idx(V, gpad), _pad_idx(V, spad)
    Eg, Es = _pad_idx(E, gpad), _pad_idx(E, spad)

    ones16 = jnp.ones((N, 16), jnp.float32)
    z16 = jnp.zeros((CHUNK, 16), jnp.float32)
    z128 = jnp.zeros((CHUNK, D_FEAT), jnp.float32)

    dp = _pass16(ones16, Vg, Vs, z16)
    dep = _pass16(ones16, Eg, Es, z16)
    rsqd, rde, xs = _prep(dp, dep, x)

    t0p = _pass128(xs, Vg, Es, z128)
    Z, c = _tcz(t0p, rde)
    cinit = c
    G = None
    for _ in range(STEPS):
        xvp = _pass128(Z, Eg, Vs, z128)
        G, Gs = _tcg(xvp, rsqd, x, c, cinit)
        xep = _pass128(Gs, Vg, Es, z128)
        Z, c = _tcz(xep, rde)
    return _mlp(G, c, W1, b1, gamma, beta, W2, b2)


# speedup vs baseline: 4.8528x; 1.1482x over previous
import functools

import jax
import jax.numpy as jnp
from jax import lax
from jax.experimental import pallas as pl
from jax.experimental.pallas import tpu as pltpu
from jax.experimental.pallas import tpu_sc as plsc

N = 10000
NE = 10000
P = 320000
D_FEAT = 128
HID = 256
OUT = 64
ALPHA = 0.1
STEPS = 10

NC, NS = 2, 16
NW = NC * NS
PPW = P // NW
CHUNK = 128
NCHUNK = 80
SLOTS = NCHUNK * CHUNK
RING = 2
HALF = NCHUNK // 2
ACC_ROWS = N + 112
RPT = ACC_ROWS // NS


def _sc_pass(feat):
    mesh = plsc.VectorSubcoreMesh(
        core_axis_name="c", subcore_axis_name="s", num_cores=NC, num_subcores=NS
    )

    @functools.partial(
        pl.kernel,
        out_type=jax.ShapeDtypeStruct((NC, ACC_ROWS, feat), jnp.float32),
        mesh=mesh,
        compiler_params=pltpu.CompilerParams(use_tc_tiling_on_sc=(feat == D_FEAT)),
        scratch_types=[
            pltpu.VMEM((HALF, CHUNK), jnp.int32),
            pltpu.VMEM((HALF, CHUNK), jnp.int32),
            [pltpu.VMEM((CHUNK, feat), jnp.float32) for _ in range(RING)],
            pltpu.VMEM_SHARED((ACC_ROWS, feat), jnp.float32),
            [pltpu.SemaphoreType.DMA for _ in range(RING)],
            [pltpu.SemaphoreType.DMA for _ in range(RING)],
        ],
    )
    def k(src, gidx, sidx, zeros, out, gidx_v, sidx_v, rows, acc, gsem, ssem):
        c = lax.axis_index("c")
        s = lax.axis_index("s")
        wid = c * NS + s
        base = s * RPT
        off = 0
        for nrows in (128, 128, 128, 128, 120):
            pltpu.sync_copy(zeros.at[pl.ds(0, nrows)], acc.at[pl.ds(base + off, nrows)])
            off += nrows
        plsc.subcore_barrier()

        def gather(j, r):
            pltpu.make_async_copy(src.at[gidx_v.at[j]], rows[r], gsem[r]).start()

        for h in range(NCHUNK // HALF):
            pltpu.sync_copy(gidx.at[wid, pl.ds(h * HALF, HALF)], gidx_v)
            pltpu.sync_copy(sidx.at[wid, pl.ds(h * HALF, HALF)], sidx_v)

            for r in range(RING):
                gather(r, r)

            @pl.loop(0, HALF // RING)
            def _(i):
                for r in range(RING):
                    j = i * RING + r
                    pltpu.make_async_copy(src.at[gidx_v.at[j]], rows[r], gsem[r]).wait()
                    pltpu.sync_copy(rows[r], acc.at[sidx_v.at[j]], add=True)

                    @pl.when(j + RING < HALF)
                    def _():
                        gather(j + RING, r)

        plsc.subcore_barrier()
        pltpu.sync_copy(acc.at[pl.ds(base, RPT)], out.at[c, pl.ds(base, RPT)])

    return k


_pass16 = _sc_pass(16)
_pass128 = _sc_pass(D_FEAT)


def _prep_body(dp_ref, dep_ref, x_ref, rsqd_ref, rde_ref, xs_ref):
    d = dp_ref[0, :N, 0:1] + dp_ref[1, :N, 0:1]
    de = dep_ref[0, :N, 0:1] + dep_ref[1, :N, 0:1]
    rsqd = lax.rsqrt(d)
    rsqd_ref[...] = rsqd
    rde_ref[...] = 1.0 / de
    xs_ref[...] = x_ref[...] * rsqd


def _prep(dp, dep, x):
    return pl.pallas_call(
        _prep_body,
        out_shape=(
            jax.ShapeDtypeStruct((N, 1), jnp.float32),
            jax.ShapeDtypeStruct((N, 1), jnp.float32),
            jax.ShapeDtypeStruct((N, D_FEAT), jnp.float32),
        ),
    )(dp, dep, x)


def _z_body(xep_ref, rde_ref, z_ref, c_ref):
    z = (xep_ref[0, :N, :] + xep_ref[1, :N, :]) * rde_ref[...]
    z_ref[...] = z
    ssq = jnp.sum(z * z)
    c_ref[...] = jnp.full((1, 1), 0.5, jnp.float32) * lax.rsqrt(ssq)


def _tcz(xep, rde):
    return pl.pallas_call(
        _z_body,
        out_shape=(
            jax.ShapeDtypeStruct((N, D_FEAT), jnp.float32),
            jax.ShapeDtypeStruct((1, 1), jnp.float32),
        ),
    )(xep, rde)


def _g_body(xvp_ref, rsqd_ref, x_ref, cprev_ref, cinit_ref, g_ref, gs_ref):
    hv = (xvp_ref[0, :N, :] + xvp_ref[1, :N, :]) * rsqd_ref[...]
    g = ((1.0 - ALPHA) * cprev_ref[0:1, 0:1]) * hv + (ALPHA * cinit_ref[0:1, 0:1]) * x_ref[...]
    g_ref[...] = g
    gs_ref[...] = g * rsqd_ref[...]


def _tcg(xvp, rsqd, x, cprev, cinit):
    return pl.pallas_call(
        _g_body,
        out_shape=(
            jax.ShapeDtypeStruct((N, D_FEAT), jnp.float32),
            jax.ShapeDtypeStruct((N, D_FEAT), jnp.float32),
        ),
    )(xvp, rsqd, x, cprev, cinit)


def _mlp_body(g_ref, c_ref, w1_ref, b1_ref, gamma_ref, beta_ref, w2_ref, b2_ref, o_ref):
    f = g_ref[...] * c_ref[0:1, 0:1]
    h = jnp.dot(f, w1_ref[...], preferred_element_type=jnp.float32) + b1_ref[...]
    mu = jnp.mean(h, axis=-1, keepdims=True)
    var = jnp.mean((h - mu) * (h - mu), axis=-1, keepdims=True)
    hn = (h - mu) * lax.rsqrt(var + 1e-5) * gamma_ref[...] + beta_ref[...]
    hr = jnp.maximum(hn, 0.0)
    o_ref[...] = jnp.dot(hr, w2_ref[...], preferred_element_type=jnp.float32) + b2_ref[...]


def _mlp(g, c, w1, b1, gamma, beta, w2, b2):
    return pl.pallas_call(
        _mlp_body,
        out_shape=jax.ShapeDtypeStruct((N, OUT), jnp.float32),
    )(g, c, w1, b1, gamma, beta, w2, b2)


def _pad_idx(idx, pad_vals):
    blocks = idx.reshape(NW, PPW)
    pad = jnp.broadcast_to(pad_vals[:, None], (NW, SLOTS - PPW)).astype(jnp.int32)
    return jnp.concatenate([blocks, pad], axis=1).reshape(NW, NCHUNK, CHUNK)


def kernel(x, hyperedge_index, W1, b1, gamma, beta, W2, b2):
    V = hyperedge_index[0]
    E = hyperedge_index[1]
    gpad = jnp.zeros((NW,), jnp.int32)
    spad = N + (jnp.arange(NW, dtype=jnp.int32) % NS)
    Vg, Vs = _pad_idx(V, gpad), _pad_idx(V, spad)
    Eg, Es = _pad_idx(E, gpad), _pad_idx(E, spad)

    ones16 = jnp.ones((N, 16), jnp.float32)
    z16 = jnp.zeros((CHUNK, 16), jnp.float32)
    z128 = jnp.zeros((CHUNK, D_FEAT), jnp.float32)

    dp = _pass16(ones16, Vg, Vs, z16)
    dep = _pass16(ones16, Eg, Es, z16)
    rsqd, rde, xs = _prep(dp, dep, x)

    t0p = _pass128(xs, Vg, Es, z128)
    Z, c = _tcz(t0p, rde)
    cinit = c
    G = None
    for _ in range(STEPS):
        xvp = _pass128(Z, Eg, Vs, z128)
        G, Gs = _tcg(xvp, rsqd, x, c, cinit)
        xep = _pass128(Gs, Vg, Es, z128)
        Z, c = _tcz(xep, rde)
    return _mlp(G, c, W1, b1, gamma, beta, W2, b2)


# speedup vs baseline: 4.8579x; 1.0011x over previous
import functools

import jax
import jax.numpy as jnp
from jax import lax
from jax.experimental import pallas as pl
from jax.experimental.pallas import tpu as pltpu
from jax.experimental.pallas import tpu_sc as plsc

N = 10000
NE = 10000
P = 320000
D_FEAT = 128
HID = 256
OUT = 64
ALPHA = 0.1
STEPS = 10

NC, NS = 2, 16
NW = NC * NS
PPW = P // NW
CHUNK = 128
NCHUNK = 80
SLOTS = NCHUNK * CHUNK
RING = 2
HALF = NCHUNK // 2
ACC_ROWS = N + 112
RPT = ACC_ROWS // NS


def _sc_pass(feat):
    mesh = plsc.VectorSubcoreMesh(
        core_axis_name="c", subcore_axis_name="s", num_cores=NC, num_subcores=NS
    )

    @functools.partial(
        pl.kernel,
        out_type=jax.ShapeDtypeStruct((NC, ACC_ROWS, feat), jnp.float32),
        mesh=mesh,
        compiler_params=pltpu.CompilerParams(use_tc_tiling_on_sc=(feat == D_FEAT)),
        scratch_types=[
            pltpu.VMEM((HALF, CHUNK), jnp.int32),
            pltpu.VMEM((HALF, CHUNK), jnp.int32),
            [pltpu.VMEM((CHUNK, feat), jnp.float32) for _ in range(RING)],
            pltpu.VMEM_SHARED((ACC_ROWS, feat), jnp.float32),
            [pltpu.SemaphoreType.DMA for _ in range(RING)],
            [pltpu.SemaphoreType.DMA for _ in range(RING)],
        ],
    )
    def k(src, gidx, sidx, zeros, out, gidx_v, sidx_v, rows, acc, gsem, ssem):
        c = lax.axis_index("c")
        s = lax.axis_index("s")
        wid = c * NS + s
        base = s * RPT
        off = 0
        for nrows in (128, 128, 128, 128, 120):
            pltpu.sync_copy(zeros.at[pl.ds(0, nrows)], acc.at[pl.ds(base + off, nrows)])
            off += nrows
        plsc.subcore_barrier()

        def gather(j, r):
            pltpu.make_async_copy(src.at[gidx_v.at[j]], rows[r], gsem[r]).start()

        for h in range(NCHUNK // HALF):
            pltpu.sync_copy(gidx.at[wid, pl.ds(h * HALF, HALF)], gidx_v)
            pltpu.sync_copy(sidx.at[wid, pl.ds(h * HALF, HALF)], sidx_v)

            for r in range(RING):
                gather(r, r)

            @pl.loop(0, HALF // RING)
            def _(i):
                for r in range(RING):
                    j = i * RING + r
                    pltpu.make_async_copy(src.at[gidx_v.at[j]], rows[r], gsem[r]).wait()
                    pltpu.make_async_copy(rows[r], acc.at[sidx_v.at[j]], ssem[r]).start(add=True)

                    @pl.when(j + RING < HALF)
                    def _():
                        pltpu.make_async_copy(rows[r], acc.at[sidx_v.at[j]], ssem[r]).wait()
                        gather(j + RING, r)

            for r in range(RING):
                pltpu.make_async_copy(rows[r], acc.at[sidx_v.at[0]], ssem[r]).wait()

        plsc.subcore_barrier()
        pltpu.sync_copy(acc.at[pl.ds(base, RPT)], out.at[c, pl.ds(base, RPT)])

    return k


_pass16 = _sc_pass(16)
_pass128 = _sc_pass(D_FEAT)


def _prep_body(dp_ref, dep_ref, x_ref, rsqd_ref, rde_ref, xs_ref):
    d = dp_ref[0, :N, 0:1] + dp_ref[1, :N, 0:1]
    de = dep_ref[0, :N, 0:1] + dep_ref[1, :N, 0:1]
    rsqd = lax.rsqrt(d)
    rsqd_ref[...] = rsqd
    rde_ref[...] = 1.0 / de
    xs_ref[...] = x_ref[...] * rsqd


def _prep(dp, dep, x):
    return pl.pallas_call(
        _prep_body,
        out_shape=(
            jax.ShapeDtypeStruct((N, 1), jnp.float32),
            jax.ShapeDtypeStruct((N, 1), jnp.float32),
            jax.ShapeDtypeStruct((N, D_FEAT), jnp.float32),
        ),
    )(dp, dep, x)


def _z_body(xep_ref, rde_ref, z_ref, c_ref):
    z = (xep_ref[0, :N, :] + xep_ref[1, :N, :]) * rde_ref[...]
    z_ref[...] = z
    ssq = jnp.sum(z * z)
    c_ref[...] = jnp.full((1, 1), 0.5, jnp.float32) * lax.rsqrt(ssq)


def _tcz(xep, rde):
    return pl.pallas_call(
        _z_body,
        out_shape=(
            jax.ShapeDtypeStruct((N, D_FEAT), jnp.float32),
            jax.ShapeDtypeStruct((1, 1), jnp.float32),
        ),
    )(xep, rde)


def _g_body(xvp_ref, rsqd_ref, x_ref, cprev_ref, cinit_ref, g_ref, gs_ref):
    hv = (xvp_ref[0, :N, :] + xvp_ref[1, :N, :]) * rsqd_ref[...]
    g = ((1.0 - ALPHA) * cprev_ref[0:1, 0:1]) * hv + (ALPHA * cinit_ref[0:1, 0:1]) * x_ref[...]
    g_ref[...] = g
    gs_ref[...] = g * rsqd_ref[...]


def _tcg(xvp, rsqd, x, cprev, cinit):
    return pl.pallas_call(
        _g_body,
        out_shape=(
            jax.ShapeDtypeStruct((N, D_FEAT), jnp.float32),
            jax.ShapeDtypeStruct((N, D_FEAT), jnp.float32),
        ),
    )(xvp, rsqd, x, cprev, cinit)


def _mlp_body(g_ref, c_ref, w1_ref, b1_ref, gamma_ref, beta_ref, w2_ref, b2_ref, o_ref):
    f = g_ref[...] * c_ref[0:1, 0:1]
    h = jnp.dot(f, w1_ref[...], preferred_element_type=jnp.float32) + b1_ref[...]
    mu = jnp.mean(h, axis=-1, keepdims=True)
    var = jnp.mean((h - mu) * (h - mu), axis=-1, keepdims=True)
    hn = (h - mu) * lax.rsqrt(var + 1e-5) * gamma_ref[...] + beta_ref[...]
    hr = jnp.maximum(hn, 0.0)
    o_ref[...] = jnp.dot(hr, w2_ref[...], preferred_element_type=jnp.float32) + b2_ref[...]


def _mlp(g, c, w1, b1, gamma, beta, w2, b2):
    return pl.pallas_call(
        _mlp_body,
        out_shape=jax.ShapeDtypeStruct((N, OUT), jnp.float32),
    )(g, c, w1, b1, gamma, beta, w2, b2)


def _pad_idx(idx, pad_vals):
    blocks = idx.reshape(NW, PPW)
    pad = jnp.broadcast_to(pad_vals[:, None], (NW, SLOTS - PPW)).astype(jnp.int32)
    return jnp.concatenate([blocks, pad], axis=1).reshape(NW, NCHUNK, CHUNK)


def kernel(x, hyperedge_index, W1, b1, gamma, beta, W2, b2):
    V = hyperedge_index[0]
    E = hyperedge_index[1]
    gpad = jnp.zeros((NW,), jnp.int32)
    spad = N + (jnp.arange(NW, dtype=jnp.int32) % NS)
    Vg, Vs = _pad_idx(V, gpad), _pad_idx(V, spad)
    Eg, Es = _pad_idx(E, gpad), _pad_idx(E, spad)

    ones16 = jnp.ones((N, 16), jnp.float32)
    z16 = jnp.zeros((CHUNK, 16), jnp.float32)
    z128 = jnp.zeros((CHUNK, D_FEAT), jnp.float32)

    dp = _pass16(ones16, Vg, Vs, z16)
    dep = _pass16(ones16, Eg, Es, z16)
    rsqd, rde, xs = _prep(dp, dep, x)

    t0p = _pass128(xs, Vg, Es, z128)
    Z, c = _tcz(t0p, rde)
    cinit = c
    G = None
    for _ in range(STEPS):
        xvp = _pass128(Z, Eg, Vs, z128)
        G, Gs = _tcg(xvp, rsqd, x, c, cinit)
        xep = _pass128(Gs, Vg, Es, z128)
        Z, c = _tcz(xep, rde)
    return _mlp(G, c, W1, b1, gamma, beta, W2, b2)


# speedup vs baseline: 4.9079x; 1.0103x over previous
import functools

import jax
import jax.numpy as jnp
from jax import lax
from jax.experimental import pallas as pl
from jax.experimental.pallas import tpu as pltpu
from jax.experimental.pallas import tpu_sc as plsc

N = 10000
NE = 10000
P = 320000
D_FEAT = 128
HID = 256
OUT = 64
ALPHA = 0.1
STEPS = 10

NC, NS = 2, 16
NW = NC * NS
PPW = P // NW
CHUNK = 128
NCHUNK = 80
SLOTS = NCHUNK * CHUNK
RING = 2
HALF = NCHUNK // 2
ACC_ROWS = N + 112
RPT = ACC_ROWS // NS
SRPT = 640

_MESH = plsc.VectorSubcoreMesh(
    core_axis_name="c", subcore_axis_name="s", num_cores=NC, num_subcores=NS
)


def _sc_pass(feat):
    @functools.partial(
        pl.kernel,
        out_type=jax.ShapeDtypeStruct((NC, ACC_ROWS, feat), jnp.float32),
        mesh=_MESH,
        compiler_params=pltpu.CompilerParams(use_tc_tiling_on_sc=(feat == D_FEAT)),
        scratch_types=[
            pltpu.VMEM((HALF, CHUNK), jnp.int32),
            pltpu.VMEM((HALF, CHUNK), jnp.int32),
            [pltpu.VMEM((CHUNK, feat), jnp.float32) for _ in range(RING)],
            pltpu.VMEM_SHARED((ACC_ROWS, feat), jnp.float32),
            [pltpu.SemaphoreType.DMA for _ in range(RING)],
            [pltpu.SemaphoreType.DMA for _ in range(RING)],
        ],
    )
    def k(src, gidx, sidx, zeros, out, gidx_v, sidx_v, rows, acc, gsem, ssem):
        c = lax.axis_index("c")
        s = lax.axis_index("s")
        wid = c * NS + s
        base = s * RPT
        off = 0
        for nrows in (128, 128, 128, 128, 120):
            pltpu.sync_copy(zeros.at[pl.ds(0, nrows)], acc.at[pl.ds(base + off, nrows)])
            off += nrows
        plsc.subcore_barrier()

        def gather(j, r):
            pltpu.make_async_copy(src.at[gidx_v.at[j]], rows[r], gsem[r]).start()

        for h in range(NCHUNK // HALF):
            pltpu.sync_copy(gidx.at[wid, pl.ds(h * HALF, HALF)], gidx_v)
            pltpu.sync_copy(sidx.at[wid, pl.ds(h * HALF, HALF)], sidx_v)

            for r in range(RING):
                gather(r, r)

            @pl.loop(0, HALF // RING)
            def _(i):
                for r in range(RING):
                    j = i * RING + r
                    pltpu.make_async_copy(src.at[gidx_v.at[j]], rows[r], gsem[r]).wait()
                    pltpu.make_async_copy(rows[r], acc.at[sidx_v.at[j]], ssem[r]).start(add=True)

                    @pl.when(j + RING < HALF)
                    def _():
                        pltpu.make_async_copy(rows[r], acc.at[sidx_v.at[j]], ssem[r]).wait()
                        gather(j + RING, r)

            for r in range(RING):
                pltpu.make_async_copy(rows[r], acc.at[sidx_v.at[0]], ssem[r]).wait()

        plsc.subcore_barrier()
        pltpu.sync_copy(acc.at[pl.ds(base, RPT)], out.at[c, pl.ds(base, RPT)])

    return k


_pass128 = _sc_pass(D_FEAT)


@functools.partial(
    pl.kernel,
    out_type=jax.ShapeDtypeStruct((NC, ACC_ROWS, 16), jnp.float32),
    mesh=_MESH,
    compiler_params=pltpu.CompilerParams(use_tc_tiling_on_sc=False),
    scratch_types=[
        pltpu.VMEM((NCHUNK, CHUNK), jnp.int32),
        pltpu.VMEM((NCHUNK, CHUNK), jnp.int32),
        [pltpu.VMEM((CHUNK, 16), jnp.float32) for _ in range(RING)],
        pltpu.VMEM_SHARED((N, 16), jnp.float32),
        pltpu.VMEM_SHARED((ACC_ROWS, 16), jnp.float32),
        [pltpu.SemaphoreType.DMA for _ in range(RING)],
        [pltpu.SemaphoreType.DMA for _ in range(RING)],
    ],
)
def _pass_deg(src, gidx, sidx, zeros, out, gidx_v, sidx_v, rows, src_sp, acc,
              gsem, ssem):
    c = lax.axis_index("c")
    s = lax.axis_index("s")
    wid = c * NS + s
    base = s * RPT
    srows = N - SRPT * (NS - 1)

    @pl.when(s < NS - 1)
    def _():
        pltpu.sync_copy(src.at[pl.ds(s * SRPT, SRPT)], src_sp.at[pl.ds(s * SRPT, SRPT)])

    @pl.when(s == NS - 1)
    def _():
        pltpu.sync_copy(src.at[pl.ds((NS - 1) * SRPT, srows)],
                        src_sp.at[pl.ds((NS - 1) * SRPT, srows)])

    off = 0
    for nrows in (128, 128, 128, 128, 120):
        pltpu.sync_copy(zeros.at[pl.ds(0, nrows)], acc.at[pl.ds(base + off, nrows)])
        off += nrows
    pltpu.sync_copy(gidx.at[wid], gidx_v)
    pltpu.sync_copy(sidx.at[wid], sidx_v)
    plsc.subcore_barrier()

    def gather(j, r):
        pltpu.make_async_copy(src_sp.at[gidx_v.at[j]], rows[r], gsem[r]).start()

    for r in range(RING):
        gather(r, r)

    @pl.loop(0, NCHUNK // RING)
    def _(i):
        for r in range(RING):
            j = i * RING + r
            pltpu.make_async_copy(src_sp.at[gidx_v.at[j]], rows[r], gsem[r]).wait()
            pltpu.make_async_copy(rows[r], acc.at[sidx_v.at[j]], ssem[r]).start(add=True)

            @pl.when(j + RING < NCHUNK)
            def _():
                pltpu.make_async_copy(rows[r], acc.at[sidx_v.at[j]], ssem[r]).wait()
                gather(j + RING, r)

    for r in range(RING):
        pltpu.make_async_copy(rows[r], acc.at[sidx_v.at[0]], ssem[r]).wait()

    plsc.subcore_barrier()
    pltpu.sync_copy(acc.at[pl.ds(base, RPT)], out.at[c, pl.ds(base, RPT)])


def _prep_body(dp_ref, dep_ref, x_ref, rsqd_ref, rde_ref, xs_ref):
    d = dp_ref[0, :N, 0:1] + dp_ref[1, :N, 0:1]
    de = dep_ref[0, :N, 0:1] + dep_ref[1, :N, 0:1]
    rsqd = lax.rsqrt(d)
    rsqd_ref[...] = rsqd
    rde_ref[...] = 1.0 / de
    xs_ref[...] = x_ref[...] * rsqd


def _prep(dp, dep, x):
    return pl.pallas_call(
        _prep_body,
        out_shape=(
            jax.ShapeDtypeStruct((N, 1), jnp.float32),
            jax.ShapeDtypeStruct((N, 1), jnp.float32),
            jax.ShapeDtypeStruct((N, D_FEAT), jnp.float32),
        ),
    )(dp, dep, x)


def _z_body(xep_ref, rde_ref, z_ref, c_ref):
    z = (xep_ref[0, :N, :] + xep_ref[1, :N, :]) * rde_ref[...]
    z_ref[...] = z
    ssq = jnp.sum(z * z)
    c_ref[...] = jnp.full((1, 1), 0.5, jnp.float32) * lax.rsqrt(ssq)


def _tcz(xep, rde):
    return pl.pallas_call(
        _z_body,
        out_shape=(
            jax.ShapeDtypeStruct((N, D_FEAT), jnp.float32),
            jax.ShapeDtypeStruct((1, 1), jnp.float32),
        ),
    )(xep, rde)


def _g_body(xvp_ref, rsqd_ref, x_ref, cprev_ref, cinit_ref, g_ref, gs_ref):
    hv = (xvp_ref[0, :N, :] + xvp_ref[1, :N, :]) * rsqd_ref[...]
    g = ((1.0 - ALPHA) * cprev_ref[0:1, 0:1]) * hv + (ALPHA * cinit_ref[0:1, 0:1]) * x_ref[...]
    g_ref[...] = g
    gs_ref[...] = g * rsqd_ref[...]


def _tcg(xvp, rsqd, x, cprev, cinit):
    return pl.pallas_call(
        _g_body,
        out_shape=(
            jax.ShapeDtypeStruct((N, D_FEAT), jnp.float32),
            jax.ShapeDtypeStruct((N, D_FEAT), jnp.float32),
        ),
        compiler_params=pltpu.CompilerParams(vmem_limit_bytes=100 * 1024 * 1024),
    )(xvp, rsqd, x, cprev, cinit)


def _mlp_body(g_ref, c_ref, w1_ref, b1_ref, gamma_ref, beta_ref, w2_ref, b2_ref, o_ref):
    f = g_ref[...] * c_ref[0:1, 0:1]
    h = jnp.dot(f, w1_ref[...], preferred_element_type=jnp.float32) + b1_ref[...]
    mu = jnp.mean(h, axis=-1, keepdims=True)
    var = jnp.mean((h - mu) * (h - mu), axis=-1, keepdims=True)
    hn = (h - mu) * lax.rsqrt(var + 1e-5) * gamma_ref[...] + beta_ref[...]
    hr = jnp.maximum(hn, 0.0)
    o_ref[...] = jnp.dot(hr, w2_ref[...], preferred_element_type=jnp.float32) + b2_ref[...]


def _mlp(g, c, w1, b1, gamma, beta, w2, b2):
    return pl.pallas_call(
        _mlp_body,
        out_shape=jax.ShapeDtypeStruct((N, OUT), jnp.float32),
    )(g, c, w1, b1, gamma, beta, w2, b2)


def _pad_idx(idx, pad_vals):
    blocks = idx.reshape(NW, PPW)
    pad = jnp.broadcast_to(pad_vals[:, None], (NW, SLOTS - PPW)).astype(jnp.int32)
    return jnp.concatenate([blocks, pad], axis=1).reshape(NW, NCHUNK, CHUNK)


def kernel(x, hyperedge_index, W1, b1, gamma, beta, W2, b2):
    V = hyperedge_index[0]
    E = hyperedge_index[1]
    gpad = jnp.zeros((NW,), jnp.int32)
    spad = N + (jnp.arange(NW, dtype=jnp.int32) % NS)
    Vg, Vs = _pad_idx(V, gpad), _pad_idx(V, spad)
    Eg, Es = _pad_idx(E, gpad), _pad_idx(E, spad)

    ones16 = jnp.ones((N, 16), jnp.float32)
    z16 = jnp.zeros((CHUNK, 16), jnp.float32)
    z128 = jnp.zeros((CHUNK, D_FEAT), jnp.float32)

    dp = _pass_deg(ones16, Vg, Vs, z16)
    dep = _pass_deg(ones16, Eg, Es, z16)
    rsqd, rde, xs = _prep(dp, dep, x)

    t0p = _pass128(xs, Vg, Es, z128)
    Z, c = _tcz(t0p, rde)
    cinit = c
    G = None
    for _ in range(STEPS):
        xvp = _pass128(Z, Eg, Vs, z128)
        G, Gs = _tcg(xvp, rsqd, x, c, cinit)
        xep = _pass128(Gs, Vg, Es, z128)
        Z, c = _tcz(xep, rde)
    return _mlp(G, c, W1, b1, gamma, beta, W2, b2)


# speedup vs baseline: 10.0832x; 2.0545x over previous
import functools

import jax
import jax.numpy as jnp
from jax import lax
from jax.experimental import pallas as pl
from jax.experimental.pallas import tpu as pltpu
from jax.experimental.pallas import tpu_sc as plsc

N = 10000
NE = 10000
P = 320000
D_FEAT = 128
HF = 64
HID = 256
OUT = 64
ALPHA = 0.1
STEPS = 10

NC, NS = 2, 16
NW = NC * NS
PPW = P // NW
CHUNK = 128
NCHUNK = 80
SLOTS = NCHUNK * CHUNK
RING = 2
ACC_ROWS = N + 112
RPT = ACC_ROWS // NS
SRPT = 640

_MESH = plsc.VectorSubcoreMesh(
    core_axis_name="c", subcore_axis_name="s", num_cores=NC, num_subcores=NS
)


def _sc_pass(feat):
    @functools.partial(
        pl.kernel,
        out_type=jax.ShapeDtypeStruct((NC, ACC_ROWS, feat), jnp.float32),
        mesh=_MESH,
        compiler_params=pltpu.CompilerParams(use_tc_tiling_on_sc=False),
        scratch_types=[
            pltpu.VMEM((NCHUNK, CHUNK), jnp.int32),
            pltpu.VMEM((NCHUNK, CHUNK), jnp.int32),
            [pltpu.VMEM((CHUNK, feat), jnp.float32) for _ in range(RING)],
            pltpu.VMEM_SHARED((N, feat), jnp.float32),
            pltpu.VMEM_SHARED((ACC_ROWS, feat), jnp.float32),
            [pltpu.SemaphoreType.DMA for _ in range(RING)],
            [pltpu.SemaphoreType.DMA for _ in range(RING)],
        ],
    )
    def k(src, gidx, sidx, zeros, out, gidx_v, sidx_v, rows, src_sp, acc,
          gsem, ssem):
        c = lax.axis_index("c")
        s = lax.axis_index("s")
        wid = c * NS + s
        base = s * RPT
        srows = N - SRPT * (NS - 1)

        @pl.when(s < NS - 1)
        def _():
            pltpu.sync_copy(src.at[pl.ds(s * SRPT, SRPT)],
                            src_sp.at[pl.ds(s * SRPT, SRPT)])

        @pl.when(s == NS - 1)
        def _():
            pltpu.sync_copy(src.at[pl.ds((NS - 1) * SRPT, srows)],
                            src_sp.at[pl.ds((NS - 1) * SRPT, srows)])

        off = 0
        for nrows in (128, 128, 128, 128, 120):
            pltpu.sync_copy(zeros.at[pl.ds(0, nrows)], acc.at[pl.ds(base + off, nrows)])
            off += nrows
        pltpu.sync_copy(gidx.at[wid], gidx_v)
        pltpu.sync_copy(sidx.at[wid], sidx_v)
        plsc.subcore_barrier()

        def gather(j, r):
            pltpu.make_async_copy(src_sp.at[gidx_v.at[j]], rows[r], gsem[r]).start()

        for r in range(RING):
            gather(r, r)

        @pl.loop(0, NCHUNK // RING)
        def _(i):
            for r in range(RING):
                j = i * RING + r
                pltpu.make_async_copy(src_sp.at[gidx_v.at[j]], rows[r], gsem[r]).wait()
                pltpu.make_async_copy(rows[r], acc.at[sidx_v.at[j]], ssem[r]).start(add=True)

                @pl.when(j + RING < NCHUNK)
                def _():
                    pltpu.make_async_copy(rows[r], acc.at[sidx_v.at[j]], ssem[r]).wait()
                    gather(j + RING, r)

        for r in range(RING):
            pltpu.make_async_copy(rows[r], acc.at[sidx_v.at[0]], ssem[r]).wait()

        plsc.subcore_barrier()
        pltpu.sync_copy(acc.at[pl.ds(base, RPT)], out.at[c, pl.ds(base, RPT)])

    return k


_pass_hf = _sc_pass(HF)
_pass_deg = _sc_pass(16)


def _prep_body(dp_ref, dep_ref, x_ref, rsqd_ref, rde_ref, xs0_ref, xs1_ref):
    d = dp_ref[0, :N, 0:1] + dp_ref[1, :N, 0:1]
    de = dep_ref[0, :N, 0:1] + dep_ref[1, :N, 0:1]
    rsqd = lax.rsqrt(d)
    rsqd_ref[...] = rsqd
    rde_ref[...] = 1.0 / de
    xs0_ref[...] = x_ref[:, :HF] * rsqd
    xs1_ref[...] = x_ref[:, HF:] * rsqd


def _prep(dp, dep, x):
    return pl.pallas_call(
        _prep_body,
        out_shape=(
            jax.ShapeDtypeStruct((N, 1), jnp.float32),
            jax.ShapeDtypeStruct((N, 1), jnp.float32),
            jax.ShapeDtypeStruct((N, HF), jnp.float32),
            jax.ShapeDtypeStruct((N, HF), jnp.float32),
        ),
    )(dp, dep, x)


def _z_body(xep0_ref, xep1_ref, rde_ref, z0_ref, z1_ref, c_ref):
    z0 = (xep0_ref[0, :N, :] + xep0_ref[1, :N, :]) * rde_ref[...]
    z1 = (xep1_ref[0, :N, :] + xep1_ref[1, :N, :]) * rde_ref[...]
    z0_ref[...] = z0
    z1_ref[...] = z1
    ssq = jnp.sum(z0 * z0) + jnp.sum(z1 * z1)
    c_ref[...] = jnp.full((1, 1), 0.5, jnp.float32) * lax.rsqrt(ssq)


def _tcz(xep0, xep1, rde):
    return pl.pallas_call(
        _z_body,
        out_shape=(
            jax.ShapeDtypeStruct((N, HF), jnp.float32),
            jax.ShapeDtypeStruct((N, HF), jnp.float32),
            jax.ShapeDtypeStruct((1, 1), jnp.float32),
        ),
    )(xep0, xep1, rde)


def _g_body(xvp0_ref, xvp1_ref, rsqd_ref, x_ref, cprev_ref, cinit_ref,
            g0_ref, g1_ref, gs0_ref, gs1_ref):
    rsqd = rsqd_ref[...]
    ca = (1.0 - ALPHA) * cprev_ref[0:1, 0:1]
    cb = ALPHA * cinit_ref[0:1, 0:1]
    g0 = ca * (xvp0_ref[0, :N, :] + xvp0_ref[1, :N, :]) * rsqd + cb * x_ref[:, :HF]
    g1 = ca * (xvp1_ref[0, :N, :] + xvp1_ref[1, :N, :]) * rsqd + cb * x_ref[:, HF:]
    g0_ref[...] = g0
    g1_ref[...] = g1
    gs0_ref[...] = g0 * rsqd
    gs1_ref[...] = g1 * rsqd


def _tcg(xvp0, xvp1, rsqd, x, cprev, cinit):
    return pl.pallas_call(
        _g_body,
        out_shape=tuple(jax.ShapeDtypeStruct((N, HF), jnp.float32) for _ in range(4)),
        compiler_params=pltpu.CompilerParams(vmem_limit_bytes=100 * 1024 * 1024),
    )(xvp0, xvp1, rsqd, x, cprev, cinit)


def _mlp_body(g0_ref, g1_ref, c_ref, w1_ref, b1_ref, gamma_ref, beta_ref,
              w2_ref, b2_ref, o_ref):
    f = jnp.concatenate([g0_ref[...], g1_ref[...]], axis=1) * c_ref[0:1, 0:1]
    h = jnp.dot(f, w1_ref[...], preferred_element_type=jnp.float32) + b1_ref[...]
    mu = jnp.mean(h, axis=-1, keepdims=True)
    var = jnp.mean((h - mu) * (h - mu), axis=-1, keepdims=True)
    hn = (h - mu) * lax.rsqrt(var + 1e-5) * gamma_ref[...] + beta_ref[...]
    hr = jnp.maximum(hn, 0.0)
    o_ref[...] = jnp.dot(hr, w2_ref[...], preferred_element_type=jnp.float32) + b2_ref[...]


def _mlp(g0, g1, c, w1, b1, gamma, beta, w2, b2):
    return pl.pallas_call(
        _mlp_body,
        out_shape=jax.ShapeDtypeStruct((N, OUT), jnp.float32),
    )(g0, g1, c, w1, b1, gamma, beta, w2, b2)


def _pad_idx(idx, pad_vals):
    blocks = idx.reshape(NW, PPW)
    pad = jnp.broadcast_to(pad_vals[:, None], (NW, SLOTS - PPW)).astype(jnp.int32)
    return jnp.concatenate([blocks, pad], axis=1).reshape(NW, NCHUNK, CHUNK)


def kernel(x, hyperedge_index, W1, b1, gamma, beta, W2, b2):
    V = hyperedge_index[0]
    E = hyperedge_index[1]
    gpad = jnp.zeros((NW,), jnp.int32)
    spad = N + (jnp.arange(NW, dtype=jnp.int32) % NS)
    Vg, Vs = _pad_idx(V, gpad), _pad_idx(V, spad)
    Eg, Es = _pad_idx(E, gpad), _pad_idx(E, spad)

    ones16 = jnp.ones((N, 16), jnp.float32)
    z16 = jnp.zeros((CHUNK, 16), jnp.float32)
    zhf = jnp.zeros((CHUNK, HF), jnp.float32)

    def pass2(s0, s1, gi, si):
        return _pass_hf(s0, gi, si, zhf), _pass_hf(s1, gi, si, zhf)

    dp = _pass_deg(ones16, Vg, Vs, z16)
    dep = _pass_deg(ones16, Eg, Es, z16)
    rsqd, rde, xs0, xs1 = _prep(dp, dep, x)

    t0p0, t0p1 = pass2(xs0, xs1, Vg, Es)
    Z0, Z1, c = _tcz(t0p0, t0p1, rde)
    cinit = c
    G0 = G1 = None
    for _ in range(STEPS):
        xvp0, xvp1 = pass2(Z0, Z1, Eg, Vs)
        G0, G1, Gs0, Gs1 = _tcg(xvp0, xvp1, rsqd, x, c, cinit)
        xep0, xep1 = pass2(Gs0, Gs1, Vg, Es)
        Z0, Z1, c = _tcz(xep0, xep1, rde)
    return _mlp(G0, G1, c, W1, b1, gamma, beta, W2, b2)


# speedup vs baseline: 10.2536x; 1.0169x over previous
import functools

import jax
import jax.numpy as jnp
from jax import lax
from jax.experimental import pallas as pl
from jax.experimental.pallas import tpu as pltpu
from jax.experimental.pallas import tpu_sc as plsc

N = 10000
NE = 10000
P = 320000
D_FEAT = 128
HF = 64
HID = 256
OUT = 64
ALPHA = 0.1
STEPS = 10

NC, NS = 2, 16
NW = NC * NS
PPW = P // NW
CHUNK = 128
NCHUNK = 80
SLOTS = NCHUNK * CHUNK
RING = 2
ACC_ROWS = N + 112
RPT = ACC_ROWS // NS
SRPT = 640

_MESH = plsc.VectorSubcoreMesh(
    core_axis_name="c", subcore_axis_name="s", num_cores=NC, num_subcores=NS
)


def _sc_pass(feat):
    @functools.partial(
        pl.kernel,
        out_type=[jax.ShapeDtypeStruct((NC, ACC_ROWS, feat), jnp.float32),
                  jax.ShapeDtypeStruct((NC, ACC_ROWS, feat), jnp.float32)],
        mesh=_MESH,
        compiler_params=pltpu.CompilerParams(use_tc_tiling_on_sc=False),
        scratch_types=[
            pltpu.VMEM((NCHUNK, CHUNK), jnp.int32),
            pltpu.VMEM((NCHUNK, CHUNK), jnp.int32),
            [pltpu.VMEM((CHUNK, feat), jnp.float32) for _ in range(RING)],
            pltpu.VMEM_SHARED((N, feat), jnp.float32),
            pltpu.VMEM_SHARED((ACC_ROWS, feat), jnp.float32),
            [pltpu.SemaphoreType.DMA for _ in range(RING)],
            [pltpu.SemaphoreType.DMA for _ in range(RING)],
        ],
    )
    def k(src0, src1, gidx, sidx, zeros, out0, out1, gidx_v, sidx_v, rows,
          src_sp, acc, gsem, ssem):
        c = lax.axis_index("c")
        s = lax.axis_index("s")
        wid = c * NS + s
        base = s * RPT
        srows = N - SRPT * (NS - 1)

        pltpu.sync_copy(gidx.at[wid], gidx_v)
        pltpu.sync_copy(sidx.at[wid], sidx_v)

        def gather(j, r):
            pltpu.make_async_copy(src_sp.at[gidx_v.at[j]], rows[r], gsem[r]).start()

        for src, out in ((src0, out0), (src1, out1)):
            @pl.when(s < NS - 1)
            def _():
                pltpu.sync_copy(src.at[pl.ds(s * SRPT, SRPT)],
                                src_sp.at[pl.ds(s * SRPT, SRPT)])

            @pl.when(s == NS - 1)
            def _():
                pltpu.sync_copy(src.at[pl.ds((NS - 1) * SRPT, srows)],
                                src_sp.at[pl.ds((NS - 1) * SRPT, srows)])

            off = 0
            for nrows in (128, 128, 128, 128, 120):
                pltpu.sync_copy(zeros.at[pl.ds(0, nrows)],
                                acc.at[pl.ds(base + off, nrows)])
                off += nrows
            plsc.subcore_barrier()

            for r in range(RING):
                gather(r, r)

            @pl.loop(0, NCHUNK // RING)
            def _(i):
                for r in range(RING):
                    j = i * RING + r
                    pltpu.make_async_copy(src_sp.at[gidx_v.at[j]], rows[r], gsem[r]).wait()
                    pltpu.make_async_copy(rows[r], acc.at[sidx_v.at[j]], ssem[r]).start(add=True)

                    @pl.when(j + RING < NCHUNK)
                    def _():
                        pltpu.make_async_copy(rows[r], acc.at[sidx_v.at[j]], ssem[r]).wait()
                        gather(j + RING, r)

            for r in range(RING):
                pltpu.make_async_copy(rows[r], acc.at[sidx_v.at[0]], ssem[r]).wait()

            plsc.subcore_barrier()
            pltpu.sync_copy(acc.at[pl.ds(base, RPT)], out.at[c, pl.ds(base, RPT)])

    return k


_pass_hf = _sc_pass(HF)


@functools.partial(
    pl.kernel,
    out_type=[jax.ShapeDtypeStruct((NC, ACC_ROWS, 16), jnp.float32),
              jax.ShapeDtypeStruct((NC, ACC_ROWS, 16), jnp.float32)],
    mesh=_MESH,
    compiler_params=pltpu.CompilerParams(use_tc_tiling_on_sc=False),
    scratch_types=[
        pltpu.VMEM((NCHUNK, CHUNK), jnp.int32),
        pltpu.VMEM((NCHUNK, CHUNK), jnp.int32),
        [pltpu.VMEM((CHUNK, 16), jnp.float32) for _ in range(RING)],
        pltpu.VMEM_SHARED((N, 16), jnp.float32),
        pltpu.VMEM_SHARED((ACC_ROWS, 16), jnp.float32),
        [pltpu.SemaphoreType.DMA for _ in range(RING)],
        [pltpu.SemaphoreType.DMA for _ in range(RING)],
    ],
)
def _pass_deg(src, gidx0, sidx0, gidx1, sidx1, zeros, out0, out1,
              gidx_v, sidx_v, rows, src_sp, acc, gsem, ssem):
    c = lax.axis_index("c")
    s = lax.axis_index("s")
    wid = c * NS + s
    base = s * RPT
    srows = N - SRPT * (NS - 1)

    @pl.when(s < NS - 1)
    def _():
        pltpu.sync_copy(src.at[pl.ds(s * SRPT, SRPT)],
                        src_sp.at[pl.ds(s * SRPT, SRPT)])

    @pl.when(s == NS - 1)
    def _():
        pltpu.sync_copy(src.at[pl.ds((NS - 1) * SRPT, srows)],
                        src_sp.at[pl.ds((NS - 1) * SRPT, srows)])

    def gather(j, r):
        pltpu.make_async_copy(src_sp.at[gidx_v.at[j]], rows[r], gsem[r]).start()

    for gidx, sidx, out in ((gidx0, sidx0, out0), (gidx1, sidx1, out1)):
        pltpu.sync_copy(gidx.at[wid], gidx_v)
        pltpu.sync_copy(sidx.at[wid], sidx_v)
        off = 0
        for nrows in (128, 128, 128, 128, 120):
            pltpu.sync_copy(zeros.at[pl.ds(0, nrows)],
                            acc.at[pl.ds(base + off, nrows)])
            off += nrows
        plsc.subcore_barrier()

        for r in range(RING):
            gather(r, r)

        @pl.loop(0, NCHUNK // RING)
        def _(i):
            for r in range(RING):
                j = i * RING + r
                pltpu.make_async_copy(src_sp.at[gidx_v.at[j]], rows[r], gsem[r]).wait()
                pltpu.make_async_copy(rows[r], acc.at[sidx_v.at[j]], ssem[r]).start(add=True)

                @pl.when(j + RING < NCHUNK)
                def _():
                    pltpu.make_async_copy(rows[r], acc.at[sidx_v.at[j]], ssem[r]).wait()
                    gather(j + RING, r)

        for r in range(RING):
            pltpu.make_async_copy(rows[r], acc.at[sidx_v.at[0]], ssem[r]).wait()

        plsc.subcore_barrier()
        pltpu.sync_copy(acc.at[pl.ds(base, RPT)], out.at[c, pl.ds(base, RPT)])


def _prep_body(dp_ref, dep_ref, x_ref, rsqd_ref, rde_ref, xs0_ref, xs1_ref):
    d = dp_ref[0, :N, 0:1] + dp_ref[1, :N, 0:1]
    de = dep_ref[0, :N, 0:1] + dep_ref[1, :N, 0:1]
    rsqd = lax.rsqrt(d)
    rsqd_ref[...] = rsqd
    rde_ref[...] = 1.0 / de
    xs0_ref[...] = x_ref[:, :HF] * rsqd
    xs1_ref[...] = x_ref[:, HF:] * rsqd


def _prep(dp, dep, x):
    return pl.pallas_call(
        _prep_body,
        out_shape=(
            jax.ShapeDtypeStruct((N, 1), jnp.float32),
            jax.ShapeDtypeStruct((N, 1), jnp.float32),
            jax.ShapeDtypeStruct((N, HF), jnp.float32),
            jax.ShapeDtypeStruct((N, HF), jnp.float32),
        ),
    )(dp, dep, x)


def _z_body(xep0_ref, xep1_ref, rde_ref, z0_ref, z1_ref, c_ref):
    z0 = (xep0_ref[0, :N, :] + xep0_ref[1, :N, :]) * rde_ref[...]
    z1 = (xep1_ref[0, :N, :] + xep1_ref[1, :N, :]) * rde_ref[...]
    z0_ref[...] = z0
    z1_ref[...] = z1
    ssq = jnp.sum(z0 * z0) + jnp.sum(z1 * z1)
    c_ref[...] = jnp.full((1, 1), 0.5, jnp.float32) * lax.rsqrt(ssq)


def _tcz(xep0, xep1, rde):
    return pl.pallas_call(
        _z_body,
        out_shape=(
            jax.ShapeDtypeStruct((N, HF), jnp.float32),
            jax.ShapeDtypeStruct((N, HF), jnp.float32),
            jax.ShapeDtypeStruct((1, 1), jnp.float32),
        ),
    )(xep0, xep1, rde)


def _g_body(xvp0_ref, xvp1_ref, rsqd_ref, x_ref, cprev_ref, cinit_ref,
            g0_ref, g1_ref, gs0_ref, gs1_ref):
    rsqd = rsqd_ref[...]
    ca = (1.0 - ALPHA) * cprev_ref[0:1, 0:1]
    cb = ALPHA * cinit_ref[0:1, 0:1]
    g0 = ca * (xvp0_ref[0, :N, :] + xvp0_ref[1, :N, :]) * rsqd + cb * x_ref[:, :HF]
    g1 = ca * (xvp1_ref[0, :N, :] + xvp1_ref[1, :N, :]) * rsqd + cb * x_ref[:, HF:]
    g0_ref[...] = g0
    g1_ref[...] = g1
    gs0_ref[...] = g0 * rsqd
    gs1_ref[...] = g1 * rsqd


def _tcg(xvp0, xvp1, rsqd, x, cprev, cinit):
    return pl.pallas_call(
        _g_body,
        out_shape=tuple(jax.ShapeDtypeStruct((N, HF), jnp.float32) for _ in range(4)),
        compiler_params=pltpu.CompilerParams(vmem_limit_bytes=100 * 1024 * 1024),
    )(xvp0, xvp1, rsqd, x, cprev, cinit)


def _mlp_body(g0_ref, g1_ref, c_ref, w1_ref, b1_ref, gamma_ref, beta_ref,
              w2_ref, b2_ref, o_ref):
    f = jnp.concatenate([g0_ref[...], g1_ref[...]], axis=1) * c_ref[0:1, 0:1]
    h = jnp.dot(f, w1_ref[...], preferred_element_type=jnp.float32) + b1_ref[...]
    mu = jnp.mean(h, axis=-1, keepdims=True)
    var = jnp.mean((h - mu) * (h - mu), axis=-1, keepdims=True)
    hn = (h - mu) * lax.rsqrt(var + 1e-5) * gamma_ref[...] + beta_ref[...]
    hr = jnp.maximum(hn, 0.0)
    o_ref[...] = jnp.dot(hr, w2_ref[...], preferred_element_type=jnp.float32) + b2_ref[...]


def _mlp(g0, g1, c, w1, b1, gamma, beta, w2, b2):
    return pl.pallas_call(
        _mlp_body,
        out_shape=jax.ShapeDtypeStruct((N, OUT), jnp.float32),
    )(g0, g1, c, w1, b1, gamma, beta, w2, b2)


def _pad_idx(idx, pad_vals):
    blocks = idx.reshape(NW, PPW)
    pad = jnp.broadcast_to(pad_vals[:, None], (NW, SLOTS - PPW)).astype(jnp.int32)
    return jnp.concatenate([blocks, pad], axis=1).reshape(NW, NCHUNK, CHUNK)


def kernel(x, hyperedge_index, W1, b1, gamma, beta, W2, b2):
    V = hyperedge_index[0]
    E = hyperedge_index[1]
    gpad = jnp.zeros((NW,), jnp.int32)
    spad = N + (jnp.arange(NW, dtype=jnp.int32) % NS)
    Vg, Vs = _pad_idx(V, gpad), _pad_idx(V, spad)
    Eg, Es = _pad_idx(E, gpad), _pad_idx(E, spad)

    ones16 = jnp.ones((N, 16), jnp.float32)
    z16 = jnp.zeros((CHUNK, 16), jnp.float32)
    zhf = jnp.zeros((CHUNK, HF), jnp.float32)

    def pass2(s0, s1, gi, si):
        return _pass_hf(s0, s1, gi, si, zhf)

    dp, dep = _pass_deg(ones16, Vg, Vs, Eg, Es, z16)
    rsqd, rde, xs0, xs1 = _prep(dp, dep, x)

    t0p0, t0p1 = pass2(xs0, xs1, Vg, Es)
    Z0, Z1, c = _tcz(t0p0, t0p1, rde)
    cinit = c
    G0 = G1 = None
    for _ in range(STEPS):
        xvp0, xvp1 = pass2(Z0, Z1, Eg, Vs)
        G0, G1, Gs0, Gs1 = _tcg(xvp0, xvp1, rsqd, x, c, cinit)
        xep0, xep1 = pass2(Gs0, Gs1, Vg, Es)
        Z0, Z1, c = _tcz(xep0, xep1, rde)
    return _mlp(G0, G1, c, W1, b1, gamma, beta, W2, b2)
